# baseline (speedup 1.0000x reference)
import functools

import jax
import jax.numpy as jnp
from jax import lax
from jax.experimental import pallas as pl
from jax.experimental.pallas import tpu as pltpu
from jax.experimental.pallas import tpu_sc as plsc

_F32 = jnp.float32
_HIGH = lax.Precision.HIGHEST


def _dot(a, b):
    return jnp.dot(a, b, precision=_HIGH, preferred_element_type=_F32)


@functools.cache
def _adj_builder(num_edges: int, n: int):
    info = plsc.get_sparse_core_info()
    nw = info.num_cores * info.num_subcores
    total = n * n
    ch = (-(-total // nw) + 15) // 16 * 16

    mesh = plsc.VectorSubcoreMesh(core_axis_name="c", subcore_axis_name="s")

    @functools.partial(
        pl.kernel,
        mesh=mesh,
        out_type=jax.ShapeDtypeStruct((nw * ch,), _F32),
        scratch_types=[
            pltpu.VMEM((num_edges,), jnp.int32),
            pltpu.VMEM((num_edges,), jnp.int32),
            pltpu.VMEM((ch,), _F32),
        ],
        compiler_params=pltpu.CompilerParams(needs_layout_passes=False),
    )
    def build(t_hbm, f_hbm, out_hbm, t_v, f_v, chunk_v):
        wid = lax.axis_index("s") * info.num_cores + lax.axis_index("c")
        base = wid * ch
        pltpu.sync_copy(t_hbm, t_v)
        pltpu.sync_copy(f_hbm, f_v)

        zero16 = jnp.zeros((16,), _F32)

        def zero_body(i, carry):
            chunk_v[pl.ds(i * 16, 16)] = zero16
            return carry

        lax.fori_loop(0, ch // 16, zero_body, 0)

        one16 = jnp.ones((16,), _F32)
        basev = jnp.full((16,), base, jnp.int32)

        def edge_body(i, carry):
            t16 = t_v[pl.ds(i * 16, 16)]
            f16 = f_v[pl.ds(i * 16, 16)]
            k = t16 * n + f16 - basev
            m = (k >= 0) & (k < ch)
            plsc.store_scatter(chunk_v, [jnp.where(m, k, 0)], one16, mask=m)
            return carry

        lax.fori_loop(0, num_edges // 16, edge_body, 0)
        pltpu.sync_copy(chunk_v, out_hbm.at[pl.ds(base, ch)])

    return build, ch


def _basis_body(nh_ref, oph_ref, ew_ref, eb_ref, b0_ref):
    eh = oph_ref.shape[1]
    ew = ew_ref[...]
    node_proj = _dot(nh_ref[...], ew)
    oph = oph_ref[...]
    p0 = _dot(oph, ew[:eh, :])
    p1 = _dot(oph, ew[eh:, :])
    b0_ref[...] = jnp.concatenate(
        [node_proj, p0, p1, eb_ref[...]], axis=0)


def _c1_body(b0_ref, w1_ref, c1_ref):
    c1_ref[...] = _dot(b0_ref[...], w1_ref[...])


def _g2f_body(w2_ref, fw_ref, g2f_ref):
    g2f_ref[...] = _dot(w2_ref[...], fw_ref[...])


def _final_body(ops_ref, adj_ref, c1_ref, g2f_ref, fw_ref, b1_ref, b2_ref,
                fb_ref, ga_ref, gb_ref, probs_ref, act_ref, logp_ref,
                ent_ref):
    ops = ops_ref[...]
    steps = ops.shape[0]
    r = c1_ref.shape[0]
    col = lax.broadcasted_iota(jnp.int32, (steps, r), 1)
    sel_lo = ((col == ops[:, 0:1] + 2) | (col == ops[:, 1:2] + 10)
              | (col == r - 1)).astype(_F32)
    r2 = lax.broadcasted_iota(jnp.int32, (2, r), 0)
    c2 = lax.broadcasted_iota(jnp.int32, (2, r), 1)
    sel_hi = ((c2 == r2) | (c2 == r - 1)).astype(_F32)
    sel = jnp.concatenate([sel_hi, sel_lo], axis=0)

    adj = adj_ref[...]
    adjs = _dot(adj, sel)
    z1 = _dot(adjs, c1_ref[...]) + b1_ref[...]
    h1 = jnp.maximum(z1, 0.0)

    v = _dot(h1, g2f_ref[...])
    w = _dot(adj, v)
    cvec = _dot(b2_ref[...], fw_ref[...]) + fb_ref[...]
    logits6 = w[2:, :] + cvec

    la = logits6[:, 0:3]
    lb = logits6[:, 3:6]

    def _smax(l):
        m = jnp.max(l, axis=1, keepdims=True)
        e = jnp.exp(l - m)
        return e / jnp.sum(e, axis=1, keepdims=True)

    pa = _smax(la) + 1e-5
    pb = _smax(lb) + 1e-5
    lpa = jnp.log(pa)
    lpb = jnp.log(pb)
    ya = lpa + ga_ref[...]
    yb = lpb + gb_ref[...]

    def _argmax3(y):
        y0, y1, y2 = y[:, 0:1], y[:, 1:2], y[:, 2:3]
        a = jnp.where(y1 > y0, 1, 0)
        return jnp.where(y2 > jnp.maximum(y0, y1), 2, a).astype(jnp.int32)

    aa = _argmax3(ya)
    ab = _argmax3(yb)
    c3 = lax.broadcasted_iota(jnp.int32, (steps, 3), 1)
    ona = (c3 == aa).astype(_F32)
    onb = (c3 == ab).astype(_F32)
    logp = jnp.sum(lpa * ona) + jnp.sum(lpb * onb)
    ent = -(jnp.sum(lpa * pa) + jnp.sum(lpb * pb))

    probs_ref[...] = jnp.concatenate([pa, pb], axis=1)
    act_ref[...] = jnp.concatenate([aa, ab], axis=1)
    logp_ref[...] = jnp.broadcast_to(logp, (1, 1))
    ent_ref[...] = jnp.broadcast_to(ent, (1, 1))


def _call(body, out_shapes):
    return pl.pallas_call(
        body,
        out_shape=out_shapes,
        compiler_params=pltpu.CompilerParams(
            vmem_limit_bytes=60 * 1024 * 1024),
    )


def kernel(ops, f_idx, t_idx, node_hidden, op_hidden, emb_attn_w,
           emb_attn_b, gc1_w, gc1_b, gc2_w, gc2_b, fc_w, fc_b):
    steps = ops.shape[0] // 2
    n = steps + 2
    feat = emb_attn_w.shape[1]
    hid = gc1_w.shape[1]
    ncls = fc_w.shape[1]
    ops2 = ops.reshape(steps, 2).astype(jnp.int32)

    build, ch = _adj_builder(2 * steps, n)
    adj_flat = build(t_idx.astype(jnp.int32), f_idx.astype(jnp.int32))
    adj = adj_flat[: n * n].reshape(n, n)

    b0 = _call(_basis_body, jax.ShapeDtypeStruct((19, feat), _F32))(
        node_hidden, op_hidden, emb_attn_w, emb_attn_b.reshape(1, -1))
    c1 = _call(_c1_body, jax.ShapeDtypeStruct((19, hid), _F32))(b0, gc1_w)
    g2f = _call(_g2f_body, jax.ShapeDtypeStruct((hid, ncls), _F32))(
        gc2_w, fc_w)

    g = jax.random.gumbel(jax.random.key(1), (2 * steps, 3), _F32)
    ga, gb = g[0::2], g[1::2]

    probs6, act2, logp, ent = _call(_final_body, [
        jax.ShapeDtypeStruct((steps, 6), _F32),
        jax.ShapeDtypeStruct((steps, 2), jnp.int32),
        jax.ShapeDtypeStruct((1, 1), _F32),
        jax.ShapeDtypeStruct((1, 1), _F32),
    ])(ops2, adj, c1, g2f, fc_w, gc1_b.reshape(1, -1),
       gc2_b.reshape(1, -1), fc_b.reshape(1, -1), ga, gb)

    action = act2.reshape(2 * steps)
    probs = probs6.reshape(2 * steps, 3)
    return action, logp[0, 0], ent[0, 0], probs
